# Initial kernel scaffold; baseline (speedup 1.0000x reference)
import jax
import jax.numpy as jnp
from jax.experimental import pallas as pl


def kernel(states, actions):
    raise NotImplementedError("write your pallas kernel here")


# speedup vs baseline: 1.5905x; 1.5905x over previous
import jax
import jax.numpy as jnp
from jax import lax
from jax.experimental import pallas as pl
from jax.experimental.pallas import tpu as pltpu

_NUM_ACTIONS = 1000
_S = 256


def _fused_body(states_ref, actions_ref, out_ref):
    out_ref[:, :_S] = states_ref[...]
    acts = actions_ref[...]
    cols = lax.broadcasted_iota(jnp.int32, (acts.shape[0], _NUM_ACTIONS), 1)
    out_ref[:, _S:] = (cols == acts).astype(jnp.float32)


def kernel(states, actions):
    B = states.shape[0]
    flat = states.reshape(B, _S)
    acts = actions.reshape(B, 1).astype(jnp.int32)
    R = 512
    grid = (B // R,)
    return pl.pallas_call(
        _fused_body,
        grid=grid,
        in_specs=[
            pl.BlockSpec((R, _S), lambda i: (i, 0)),
            pl.BlockSpec((R, 1), lambda i: (i, 0)),
        ],
        out_specs=pl.BlockSpec((R, _S + _NUM_ACTIONS), lambda i: (i, 0)),
        out_shape=jax.ShapeDtypeStruct((B, _S + _NUM_ACTIONS), jnp.float32),
        compiler_params=pltpu.CompilerParams(
            dimension_semantics=("arbitrary",),
        ),
    )(flat, acts)


# speedup vs baseline: 7.0341x; 4.4227x over previous
import jax
import jax.numpy as jnp
from jax import lax
from jax.experimental import pallas as pl
from jax.experimental.pallas import tpu as pltpu

_NUM_ACTIONS = 1000
_S = 256
_ROW = _S + _NUM_ACTIONS


def _t_body(flat_t_ref, actions_ref, out_ref):
    C = out_ref.shape[1]
    out_ref[:_S, :] = flat_t_ref[...]
    acts = actions_ref[...]
    rows = lax.broadcasted_iota(jnp.int32, (_NUM_ACTIONS, C), 0)
    out_ref[_S:, :] = (rows == acts).astype(jnp.float32)


def kernel(states, actions):
    B = states.shape[0]
    flat_t = states.reshape(B, _S).T
    acts_row = actions.reshape(1, B).astype(jnp.int32)
    C = 2048
    out_t = pl.pallas_call(
        _t_body,
        grid=(B // C,),
        in_specs=[
            pl.BlockSpec((_S, C), lambda i: (0, i)),
            pl.BlockSpec((1, C), lambda i: (0, i)),
        ],
        out_specs=pl.BlockSpec((_ROW, C), lambda i: (0, i)),
        out_shape=jax.ShapeDtypeStruct((_ROW, B), jnp.float32),
        compiler_params=pltpu.CompilerParams(dimension_semantics=("arbitrary",)),
    )(flat_t, acts_row)
    return out_t.T
